# Initial kernel scaffold; baseline (speedup 1.0000x reference)
import jax
import jax.numpy as jnp
from jax.experimental import pallas as pl


def kernel(x, edge_attr, group_emb, hemi_emb, conn_emb, ln_weight, ln_bias, edge_index, batch):
    raise NotImplementedError("write your pallas kernel here")


# speedup vs baseline: 3.1237x; 3.1237x over previous
import jax
import jax.numpy as jnp
from jax.experimental import pallas as pl

N = 65536
E = 1048576
G = 116
CX = 116
CP = 32
CF = 58
C = 206
EPS = 1e-5

R = 2048
EB = 2048

_INTERPRET = False


def _asym_p(xb):
    ax = jnp.abs(xb)
    x1 = jnp.concatenate([xb[:, 1:], xb[:, :1]], axis=1)
    ax1 = jnp.concatenate([ax[:, 1:], ax[:, :1]], axis=1)
    return (ax + ax1) * jnp.abs(xb - x1)


def _even_mask(r):
    lane = jax.lax.broadcasted_iota(jnp.int32, (r, CX), 1)
    return (lane % 2) == 0


def _stats_kernel(x_ref, b_ref, o_ref):
    i = pl.program_id(0)
    xb = x_ref[...]
    bb = b_ref[...]
    p = _asym_p(xb)
    pm = jnp.where(_even_mask(xb.shape[0]), p, 0.0)
    s = jnp.sum(xb, axis=1, keepdims=True) + jnp.sum(pm, axis=1, keepdims=True)
    q = jnp.sum(xb * xb, axis=1, keepdims=True) + jnp.sum(pm * pm, axis=1, keepdims=True)
    one = jnp.ones_like(s)
    zero = jnp.zeros((xb.shape[0], 5), jnp.float32)
    v = jnp.concatenate([s, q, one, zero], axis=1)
    gl = jax.lax.broadcasted_iota(jnp.int32, (xb.shape[0], 128), 1)
    onehot = (gl == bb).astype(jnp.float32)
    contrib = jax.lax.dot_general(
        onehot, v, (((0,), (0,)), ((), ())),
        preferred_element_type=jnp.float32)

    @pl.when(i == 0)
    def _():
        o_ref[...] = jnp.zeros_like(o_ref)

    o_ref[...] += contrib


def _norm_kernel(x_ref, b_ref, st_ref, pre_ref, w_ref, bias_ref, o_ref):
    xb = x_ref[...]
    bb = b_ref[...]
    r = xb.shape[0]
    p = _asym_p(xb)
    pm = jnp.where(_even_mask(r), p, 0.0)
    ci = jax.lax.broadcasted_iota(jnp.int32, (CX, CF), 0)
    ji = jax.lax.broadcasted_iota(jnp.int32, (CX, CF), 1)
    pc = (ci == 2 * ji).astype(jnp.float32)
    featc = jax.lax.dot_general(
        pm, pc, (((1,), (0,)), ((), ())),
        preferred_element_type=jnp.float32)

    pre_t = pre_ref[...]
    sp = jnp.sum(pre_t, axis=1, keepdims=True)
    qp = jnp.sum(pre_t * pre_t, axis=1, keepdims=True)
    st = st_ref[...]
    scol = st[:G, 0:1]
    qcol = st[:G, 1:2]
    cnt = jnp.maximum(st[:G, 2:3], 1.0)
    norm = cnt * float(C)
    mean = (scol + cnt * sp) / norm
    msq = (qcol + cnt * qp) / norm
    var = msq - mean * mean
    rstd = jax.lax.rsqrt(var + EPS)
    tbl = jnp.concatenate([mean, rstd, pre_t], axis=1)

    gl = jax.lax.broadcasted_iota(jnp.int32, (r, G), 1)
    onehot = (gl == bb).astype(jnp.float32)
    gath = jax.lax.dot_general(
        onehot, tbl, (((1,), (0,)), ((), ())),
        preferred_element_type=jnp.float32)
    mu = gath[:, 0:1]
    rs = gath[:, 1:2]
    prei = gath[:, 2:34]
    xc = jnp.concatenate([xb, prei, featc], axis=1)
    out = (xc - mu) * rs * w_ref[...] + bias_ref[...]
    o_ref[...] = out


def _edge_kernel(s_ref, d_ref, a_ref, ce_ref, o_ref):
    s = s_ref[...]
    d = d_ref[...]
    sp = jnp.bitwise_and(s, 1)
    dp = jnp.bitwise_and(d, 1)
    homo = (sp == 0) & (d == s + 1)
    inter = (sp != dp) & jnp.logical_not(homo)
    ct = jnp.where(homo, 0, jnp.where(inter, 1, 2))
    ce = ce_ref[...]
    cemb = jnp.where(ct == 0, ce[0:1, :],
                     jnp.where(ct == 1, ce[1:2, :], ce[2:3, :]))
    o_ref[...] = jnp.concatenate([a_ref[...], cemb], axis=1)


def kernel(x, edge_attr, group_emb, hemi_emb, conn_emb, ln_weight, ln_bias, edge_index, batch):
    group_ids = jnp.repeat(jnp.arange(4, dtype=jnp.int32), 29)
    hemi_ids = jnp.arange(G, dtype=jnp.int32) % 2
    pre = jnp.concatenate([jnp.take(group_emb, group_ids, axis=0),
                           jnp.take(hemi_emb, hemi_ids, axis=0)], axis=-1)

    b2 = batch.reshape(N, 1)
    nb = N // R
    stats = pl.pallas_call(
        _stats_kernel,
        grid=(nb,),
        in_specs=[
            pl.BlockSpec((R, CX), lambda i: (i, 0)),
            pl.BlockSpec((R, 1), lambda i: (i, 0)),
        ],
        out_specs=pl.BlockSpec((128, 8), lambda i: (0, 0)),
        out_shape=jax.ShapeDtypeStruct((128, 8), jnp.float32),
        interpret=_INTERPRET,
    )(x, b2)

    out = pl.pallas_call(
        _norm_kernel,
        grid=(nb,),
        in_specs=[
            pl.BlockSpec((R, CX), lambda i: (i, 0)),
            pl.BlockSpec((R, 1), lambda i: (i, 0)),
            pl.BlockSpec((128, 8), lambda i: (0, 0)),
            pl.BlockSpec((G, CP), lambda i: (0, 0)),
            pl.BlockSpec((1, C), lambda i: (0, 0)),
            pl.BlockSpec((1, C), lambda i: (0, 0)),
        ],
        out_specs=pl.BlockSpec((R, C), lambda i: (i, 0)),
        out_shape=jax.ShapeDtypeStruct((N, C), jnp.float32),
        interpret=_INTERPRET,
    )(x, b2, stats, pre, ln_weight.reshape(1, C), ln_bias.reshape(1, C))

    src = edge_index[0].reshape(E, 1)
    dst = edge_index[1].reshape(E, 1)
    at2 = edge_attr.reshape(E, 1)
    neb = E // EB
    eattr = pl.pallas_call(
        _edge_kernel,
        grid=(neb,),
        in_specs=[
            pl.BlockSpec((EB, 1), lambda i: (i, 0)),
            pl.BlockSpec((EB, 1), lambda i: (i, 0)),
            pl.BlockSpec((EB, 1), lambda i: (i, 0)),
            pl.BlockSpec((3, 8), lambda i: (0, 0)),
        ],
        out_specs=pl.BlockSpec((EB, 9), lambda i: (i, 0)),
        out_shape=jax.ShapeDtypeStruct((E, 9), jnp.float32),
        interpret=_INTERPRET,
    )(src, dst, at2, conn_emb)

    return out, eattr
